# baseline (speedup 1.0000x reference)
import dataclasses
import functools

import jax
import jax.numpy as jnp
from jax import lax
from jax.experimental import pallas as pl
from jax.experimental.pallas import tpu as pltpu
from jax.experimental.pallas import tpu_sc as plsc

NC = 2
NS = 16
NW = NC * NS
L = 16


@functools.partial(jax.jit, static_argnames=())
def kernel(user, rsid, user_table, rsid_table):
    B = user.shape[0]
    D = user_table.shape[1]
    b_per_w = B // NW
    C = 128
    mesh = plsc.VectorSubcoreMesh(core_axis_name="c", subcore_axis_name="s")

    n_chunks = b_per_w // C

    def body(user_hbm, rsid_hbm, ut_hbm, rt_hbm, out_hbm,
             uidx, ridx, urows0, vrows0, urows1, vrows1, outv, partials,
             sem_u0, sem_v0, sem_u1, sem_v1):
        wid = lax.axis_index("s") * NC + lax.axis_index("c")
        base = wid * b_per_w
        rest = b_per_w - C
        ci0 = pltpu.async_copy(
            user_hbm.at[pl.ds(base, C)], uidx.at[pl.ds(0, C)], sem_u0)
        cj0 = pltpu.async_copy(
            rsid_hbm.at[pl.ds(base, C)], ridx.at[pl.ds(0, C)], sem_v0)
        ci1 = pltpu.async_copy(
            user_hbm.at[pl.ds(base + C, rest)], uidx.at[pl.ds(C, rest)], sem_u1)
        cj1 = pltpu.async_copy(
            rsid_hbm.at[pl.ds(base + C, rest)], ridx.at[pl.ds(C, rest)], sem_v1)

        ubuf = (urows0, urows1)
        vbuf = (vrows0, vrows1)
        sems = ((sem_u0, sem_v0), (sem_u1, sem_v1))

        def issue(i, k):
            cu = pltpu.async_copy(
                ut_hbm.at[uidx.at[pl.ds(i * C, C)]], ubuf[k], sems[k][0])
            cv = pltpu.async_copy(
                rt_hbm.at[ridx.at[pl.ds(i * C, C)]], vbuf[k], sems[k][1])
            return cu, cv

        def _tree_sum(vals):
            while len(vals) > 1:
                vals = [a + b for a, b in zip(vals[::2], vals[1::2])]
            return vals[0]

        def compute(c0, ur, vr):
            @plsc.parallel_loop(0, C, unroll=8)
            def _row(r):
                ms = [ur[r, pl.ds(j * L, L)] * vr[r, pl.ds(j * L, L)]
                      for j in range(D // L)]
                partials[pl.ds(r * 17, L)] = _tree_sum(ms)

            @plsc.parallel_loop(0, C, step=L, unroll=2)
            def _grp(r0):
                idx0 = (lax.iota(jnp.int32, L) + r0) * 17
                vals = [plsc.load_gather(partials, [idx0 + l])
                        for l in range(L)]
                outv[pl.ds(c0 + r0, L)] = _tree_sum(vals)

        def issue_dyn(i, k):
            off = jnp.minimum(i, n_chunks - 1) * C
            cu = pltpu.async_copy(
                ut_hbm.at[uidx.at[pl.ds(off, C)]], ubuf[k], sems[k][0])
            cv = pltpu.async_copy(
                rt_hbm.at[ridx.at[pl.ds(off, C)]], vbuf[k], sems[k][1])
            return cu, cv

        ci0.wait()
        cj0.wait()
        first = issue(0, 0)
        ci1.wait()
        cj1.wait()
        first[0].wait()
        first[1].wait()

        @pl.loop(0, n_chunks, step=2)
        def _pair(i):
            nxt = issue_dyn(i + 1, 1)
            compute(i * C, ubuf[0], vbuf[0])
            nxt[0].wait()
            nxt[1].wait()
            nxt2 = issue_dyn(i + 2, 0)
            compute((i + 1) * C, ubuf[1], vbuf[1])
            nxt2[0].wait()
            nxt2[1].wait()

        pltpu.sync_copy(outv, out_hbm.at[pl.ds(base, b_per_w)])

    cp = pltpu.CompilerParams()
    if "needs_layout_passes" in pltpu.CompilerParams.__dataclass_fields__:
        cp = dataclasses.replace(cp, needs_layout_passes=False)

    kern = pl.kernel(
        body,
        out_type=jax.ShapeDtypeStruct((B,), jnp.float32),
        mesh=mesh,
        compiler_params=cp,
        scratch_types=[
            pltpu.VMEM((b_per_w,), jnp.int32),
            pltpu.VMEM((b_per_w,), jnp.int32),
            pltpu.VMEM((C, D), jnp.float32),
            pltpu.VMEM((C, D), jnp.float32),
            pltpu.VMEM((C, D), jnp.float32),
            pltpu.VMEM((C, D), jnp.float32),
            pltpu.VMEM((b_per_w,), jnp.float32),
            pltpu.VMEM((C * 17,), jnp.float32),
            pltpu.SemaphoreType.DMA,
            pltpu.SemaphoreType.DMA,
            pltpu.SemaphoreType.DMA,
            pltpu.SemaphoreType.DMA,
        ],
    )
    return kern(user.astype(jnp.int32), rsid.astype(jnp.int32),
                user_table, rsid_table)


# speedup vs baseline: 1.0410x; 1.0410x over previous
import dataclasses
import functools

import jax
import jax.numpy as jnp
from jax import lax
from jax.experimental import pallas as pl
from jax.experimental.pallas import tpu as pltpu
from jax.experimental.pallas import tpu_sc as plsc

NC = 2
NS = 16
NW = NC * NS
L = 16


@functools.partial(jax.jit, static_argnames=())
def kernel(user, rsid, user_table, rsid_table):
    B = user.shape[0]
    D = user_table.shape[1]
    b_per_w = B // NW
    C = 128
    mesh = plsc.VectorSubcoreMesh(core_axis_name="c", subcore_axis_name="s")

    n_chunks = b_per_w // C

    def body(user_hbm, rsid_hbm, ut_hbm, rt_hbm, out_hbm,
             uidx, ridx, urows0, vrows0, urows1, vrows1, outv, partials,
             sem_u0, sem_v0, sem_u1, sem_v1):
        wid = lax.axis_index("s") * NC + lax.axis_index("c")
        base = wid * b_per_w
        rest = b_per_w - C
        ci0 = pltpu.async_copy(
            user_hbm.at[pl.ds(base, C)], uidx.at[pl.ds(0, C)], sem_u0)
        cj0 = pltpu.async_copy(
            rsid_hbm.at[pl.ds(base, C)], ridx.at[pl.ds(0, C)], sem_v0)
        ci1 = pltpu.async_copy(
            user_hbm.at[pl.ds(base + C, rest)], uidx.at[pl.ds(C, rest)], sem_u1)
        cj1 = pltpu.async_copy(
            rsid_hbm.at[pl.ds(base + C, rest)], ridx.at[pl.ds(C, rest)], sem_v1)

        ubuf = (urows0, urows1)
        vbuf = (vrows0, vrows1)
        sems = ((sem_u0, sem_v0), (sem_u1, sem_v1))

        def issue(i, k):
            cu = pltpu.async_copy(
                ut_hbm.at[uidx.at[pl.ds(i * C, C)]], ubuf[k], sems[k][0])
            cv = pltpu.async_copy(
                rt_hbm.at[ridx.at[pl.ds(i * C, C)]], vbuf[k], sems[k][1])
            return cu, cv

        def _tree_sum(vals):
            while len(vals) > 1:
                vals = [a + b for a, b in zip(vals[::2], vals[1::2])]
            return vals[0]

        def compute(c0, ur, vr):
            @plsc.parallel_loop(0, C, unroll=2)
            def _row(r):
                ms = [ur[r, pl.ds(j * L, L)] * vr[r, pl.ds(j * L, L)]
                      for j in range(D // L)]
                partials[pl.ds(r * 17, L)] = _tree_sum(ms)

            @plsc.parallel_loop(0, C, step=L, unroll=2)
            def _grp(r0):
                idx0 = (lax.iota(jnp.int32, L) + r0) * 17
                vals = [plsc.load_gather(partials, [idx0 + l])
                        for l in range(L)]
                outv[pl.ds(c0 + r0, L)] = _tree_sum(vals)

        def issue_dyn(i, k):
            off = jnp.minimum(i, n_chunks - 1) * C
            cu = pltpu.async_copy(
                ut_hbm.at[uidx.at[pl.ds(off, C)]], ubuf[k], sems[k][0])
            cv = pltpu.async_copy(
                rt_hbm.at[ridx.at[pl.ds(off, C)]], vbuf[k], sems[k][1])
            return cu, cv

        ci0.wait()
        cj0.wait()
        first = issue(0, 0)
        ci1.wait()
        cj1.wait()
        first[0].wait()
        first[1].wait()

        @pl.loop(0, n_chunks, step=2)
        def _pair(i):
            nxt = issue_dyn(i + 1, 1)
            compute(i * C, ubuf[0], vbuf[0])
            nxt[0].wait()
            nxt[1].wait()
            nxt2 = issue_dyn(i + 2, 0)
            compute((i + 1) * C, ubuf[1], vbuf[1])
            nxt2[0].wait()
            nxt2[1].wait()

        pltpu.sync_copy(outv, out_hbm.at[pl.ds(base, b_per_w)])

    cp = pltpu.CompilerParams()
    if "needs_layout_passes" in pltpu.CompilerParams.__dataclass_fields__:
        cp = dataclasses.replace(cp, needs_layout_passes=False)

    kern = pl.kernel(
        body,
        out_type=jax.ShapeDtypeStruct((B,), jnp.float32),
        mesh=mesh,
        compiler_params=cp,
        scratch_types=[
            pltpu.VMEM((b_per_w,), jnp.int32),
            pltpu.VMEM((b_per_w,), jnp.int32),
            pltpu.VMEM((C, D), jnp.float32),
            pltpu.VMEM((C, D), jnp.float32),
            pltpu.VMEM((C, D), jnp.float32),
            pltpu.VMEM((C, D), jnp.float32),
            pltpu.VMEM((b_per_w,), jnp.float32),
            pltpu.VMEM((C * 17,), jnp.float32),
            pltpu.SemaphoreType.DMA,
            pltpu.SemaphoreType.DMA,
            pltpu.SemaphoreType.DMA,
            pltpu.SemaphoreType.DMA,
        ],
    )
    return kern(user.astype(jnp.int32), rsid.astype(jnp.int32),
                user_table, rsid_table)
